# Initial kernel scaffold; baseline (speedup 1.0000x reference)
import jax
import jax.numpy as jnp
from jax.experimental import pallas as pl


def kernel(input_ids, token_type_ids, word_embeddings, position_embeddings, token_type_embeddings, ln_gamma, ln_beta):
    raise NotImplementedError("write your pallas kernel here")


# speedup vs baseline: 3.0687x; 3.0687x over previous
import functools

import jax
import jax.numpy as jnp
from jax import lax
from jax.experimental import pallas as pl
from jax.experimental.pallas import tpu as pltpu
from jax.experimental.pallas import tpu_sc as plsc

B, S, V, H, P, T = 4096, 200, 100000, 128, 512, 2

NC, NS, L = 2, 16, 16
NW = NC * NS
ROWS_PER_W = B // NW
NH = H // L


def _lane_sum(x):
    lanes = jnp.arange(L, dtype=jnp.int32)
    dnums = lax.GatherDimensionNumbers(
        offset_dims=(), collapsed_slice_dims=(0,), start_index_map=(0,))
    for d in (8, 4, 2, 1):
        x = x + lax.gather(x, (lanes ^ d)[:, None], dnums, slice_sizes=(1,),
                           mode=lax.GatherScatterMode.PROMISE_IN_BOUNDS)
    return x


def _rsqrt(x):
    i = lax.bitcast_convert_type(x, jnp.int32)
    i = jnp.int32(0x5F3759DF) - (i >> 1)
    y = lax.bitcast_convert_type(i, jnp.float32)
    for _ in range(3):
        y = y * (1.5 - 0.5 * x * y * y)
    return y


def _sc_body(ids_hbm, tt_hbm, word_hbm, pos_hbm, typ_hbm, g_hbm, b_hbm,
             out_hbm, rows_v, pos_v, typ_v, g_v, b_v, idx_a, idx_b, tt_v,
             sem):
    wid = lax.axis_index("s") * NC + lax.axis_index("c")
    b0 = wid * ROWS_PER_W

    pltpu.sync_copy(pos_hbm.at[pl.ds(0, S)], pos_v)
    pltpu.sync_copy(typ_hbm, typ_v)
    pltpu.sync_copy(g_hbm, g_v)
    pltpu.sync_copy(b_hbm, b_v)

    def row_body(i, carry):
        b = b0 + i
        base = b * S
        pltpu.sync_copy(ids_hbm.at[pl.ds(base, 128)], idx_a)
        pltpu.sync_copy(ids_hbm.at[pl.ds(base + 128, 72)], idx_b)
        pltpu.sync_copy(tt_hbm.at[pl.ds(base, S)], tt_v.at[pl.ds(0, S)])
        c1 = pltpu.async_copy(word_hbm.at[idx_a], rows_v.at[pl.ds(0, 128)], sem)
        c2 = pltpu.async_copy(word_hbm.at[idx_b], rows_v.at[pl.ds(128, 72)], sem)
        c1.wait()
        c2.wait()

        def tok_body(s, c):
            tt = tt_v[pl.ds(s, L)][0]
            xs = []
            acc_s = None
            acc_q = None
            for j in range(NH):
                sl = pl.ds(j * L, L)
                x = rows_v[s, sl] + pos_v[s, sl] + typ_v[tt, sl]
                xs.append(x)
                acc_s = x if acc_s is None else acc_s + x
                acc_q = x * x if acc_q is None else acc_q + x * x
            mean = _lane_sum(acc_s) * (1.0 / H)
            ex2 = _lane_sum(acc_q) * (1.0 / H)
            inv = _rsqrt(ex2 - mean * mean + 1e-12)
            for j in range(NH):
                sl = pl.ds(j * L, L)
                rows_v[s, sl] = (xs[j] - mean) * inv * g_v[sl] + b_v[sl]
            return c

        lax.fori_loop(0, S, tok_body, 0)
        pltpu.sync_copy(rows_v, out_hbm.at[b])
        return carry

    lax.fori_loop(0, ROWS_PER_W, row_body, 0)


def kernel(input_ids, token_type_ids, word_embeddings, position_embeddings,
           token_type_embeddings, ln_gamma, ln_beta):
    mesh = plsc.VectorSubcoreMesh(core_axis_name="c", subcore_axis_name="s")
    f = pl.kernel(
        _sc_body,
        out_type=jax.ShapeDtypeStruct((B, S, H), jnp.float32),
        mesh=mesh,
        scratch_types=[
            pltpu.VMEM((S, H), jnp.float32),
            pltpu.VMEM((S, H), jnp.float32),
            pltpu.VMEM((T, H), jnp.float32),
            pltpu.VMEM((H,), jnp.float32),
            pltpu.VMEM((H,), jnp.float32),
            pltpu.VMEM((128,), jnp.int32),
            pltpu.VMEM((72,), jnp.int32),
            pltpu.VMEM((S + L,), jnp.int32),
            pltpu.SemaphoreType.DMA,
        ],
    )
    return f(input_ids.astype(jnp.int32).reshape(-1),
             token_type_ids.astype(jnp.int32).reshape(-1),
             word_embeddings, position_embeddings, token_type_embeddings,
             ln_gamma, ln_beta)


# speedup vs baseline: 5.4022x; 1.7604x over previous
import functools

import jax
import jax.numpy as jnp
from jax import lax
from jax.experimental import pallas as pl
from jax.experimental.pallas import tpu as pltpu
from jax.experimental.pallas import tpu_sc as plsc

B, S, V, H, P, T = 4096, 200, 100000, 128, 512, 2

NC, NS, L = 2, 16, 16
NW = NC * NS
ROWS_PER_W = B // NW
NH = H // L


def _lane_sum(x):
    lanes = jnp.arange(L, dtype=jnp.int32)
    dnums = lax.GatherDimensionNumbers(
        offset_dims=(), collapsed_slice_dims=(0,), start_index_map=(0,))
    for d in (8, 4, 2, 1):
        x = x + lax.gather(x, (lanes ^ d)[:, None], dnums, slice_sizes=(1,),
                           mode=lax.GatherScatterMode.PROMISE_IN_BOUNDS)
    return x


def _rsqrt(x):
    i = lax.bitcast_convert_type(x, jnp.int32)
    i = jnp.int32(0x5F3759DF) - (i >> 1)
    y = lax.bitcast_convert_type(i, jnp.float32)
    for _ in range(3):
        y = y * (1.5 - 0.5 * x * y * y)
    return y


def _sc_body(ids_hbm, tt_hbm, word_hbm, pos_hbm, typ_hbm, g_hbm, b_hbm,
             out_hbm, rows_v, pos_v, typ_v, g_v, b_v, idx_a, idx_b, tt_v,
             sem):
    wid = lax.axis_index("s") * NC + lax.axis_index("c")
    b0 = wid * ROWS_PER_W

    pltpu.sync_copy(pos_hbm.at[pl.ds(0, S)], pos_v)
    pltpu.sync_copy(typ_hbm, typ_v)
    pltpu.sync_copy(g_hbm, g_v)
    pltpu.sync_copy(b_hbm, b_v)

    def row_body(i, carry):
        b = b0 + i
        base = b * S
        pltpu.sync_copy(ids_hbm.at[pl.ds(base, 128)], idx_a)
        pltpu.sync_copy(ids_hbm.at[pl.ds(base + 128, 72)], idx_b)
        pltpu.sync_copy(tt_hbm.at[pl.ds(base, S)], tt_v.at[pl.ds(0, S)])
        c1 = pltpu.async_copy(word_hbm.at[idx_a], rows_v.at[pl.ds(0, 128)], sem)
        c2 = pltpu.async_copy(word_hbm.at[idx_b], rows_v.at[pl.ds(128, 72)], sem)
        c1.wait()
        c2.wait()

        @plsc.parallel_loop(0, S, unroll=4)
        def tok_body(s):
            tt = tt_v[pl.ds(s, L)][0]
            xs = []
            acc_s = None
            acc_q = None
            for j in range(NH):
                sl = pl.ds(j * L, L)
                x = rows_v[s, sl] + pos_v[s, sl] + typ_v[tt, sl]
                xs.append(x)
                acc_s = x if acc_s is None else acc_s + x
                acc_q = x * x if acc_q is None else acc_q + x * x
            mean = _lane_sum(acc_s) * (1.0 / H)
            ex2 = _lane_sum(acc_q) * (1.0 / H)
            inv = _rsqrt(ex2 - mean * mean + 1e-12)
            for j in range(NH):
                sl = pl.ds(j * L, L)
                rows_v[s, sl] = (xs[j] - mean) * inv * g_v[sl] + b_v[sl]
        pltpu.sync_copy(rows_v, out_hbm.at[b])
        return carry

    lax.fori_loop(0, ROWS_PER_W, row_body, 0)


def kernel(input_ids, token_type_ids, word_embeddings, position_embeddings,
           token_type_embeddings, ln_gamma, ln_beta):
    mesh = plsc.VectorSubcoreMesh(core_axis_name="c", subcore_axis_name="s")
    f = pl.kernel(
        _sc_body,
        out_type=jax.ShapeDtypeStruct((B, S, H), jnp.float32),
        mesh=mesh,
        scratch_types=[
            pltpu.VMEM((S, H), jnp.float32),
            pltpu.VMEM((S, H), jnp.float32),
            pltpu.VMEM((T, H), jnp.float32),
            pltpu.VMEM((H,), jnp.float32),
            pltpu.VMEM((H,), jnp.float32),
            pltpu.VMEM((128,), jnp.int32),
            pltpu.VMEM((72,), jnp.int32),
            pltpu.VMEM((S + L,), jnp.int32),
            pltpu.SemaphoreType.DMA,
        ],
    )
    return f(input_ids.astype(jnp.int32).reshape(-1),
             token_type_ids.astype(jnp.int32).reshape(-1),
             word_embeddings, position_embeddings, token_type_embeddings,
             ln_gamma, ln_beta)


# speedup vs baseline: 7.0575x; 1.3064x over previous
import functools

import jax
import jax.numpy as jnp
from jax import lax
from jax.experimental import pallas as pl
from jax.experimental.pallas import tpu as pltpu
from jax.experimental.pallas import tpu_sc as plsc

B, S, V, H, P, T = 4096, 200, 100000, 128, 512, 2

NC, NS, L = 2, 16, 16
NW = NC * NS
ROWS_PER_W = B // NW
NH = H // L


def _lane_sum(x):
    lanes = jnp.arange(L, dtype=jnp.int32)
    dnums = lax.GatherDimensionNumbers(
        offset_dims=(), collapsed_slice_dims=(0,), start_index_map=(0,))
    for d in (8, 4, 2, 1):
        x = x + lax.gather(x, (lanes ^ d)[:, None], dnums, slice_sizes=(1,),
                           mode=lax.GatherScatterMode.PROMISE_IN_BOUNDS)
    return x


def _rsqrt(x):
    i = lax.bitcast_convert_type(x, jnp.int32)
    i = jnp.int32(0x5F3759DF) - (i >> 1)
    y = lax.bitcast_convert_type(i, jnp.float32)
    for _ in range(3):
        y = y * (1.5 - 0.5 * x * y * y)
    return y


def _sc_body(ids_hbm, tt_hbm, word_hbm, pos_hbm, typ_hbm, g_hbm, b_hbm,
             out_hbm, rows_v, pos_v, typ_v, g_v, b_v, idx_a, idx_b, tt_v,
             sem):
    wid = lax.axis_index("s") * NC + lax.axis_index("c")
    b0 = wid * ROWS_PER_W

    pltpu.sync_copy(pos_hbm.at[pl.ds(0, S)], pos_v)
    pltpu.sync_copy(typ_hbm, typ_v)
    pltpu.sync_copy(g_hbm, g_v)
    pltpu.sync_copy(b_hbm, b_v)

    def row_body(i, carry):
        b = b0 + i
        base = b * S
        pltpu.sync_copy(ids_hbm.at[pl.ds(base, 128)], idx_a)
        pltpu.sync_copy(ids_hbm.at[pl.ds(base + 128, 72)], idx_b)
        pltpu.sync_copy(tt_hbm.at[pl.ds(base, S)], tt_v.at[pl.ds(0, S)])
        c1 = pltpu.async_copy(word_hbm.at[idx_a], rows_v.at[pl.ds(0, 128)], sem)
        c2 = pltpu.async_copy(word_hbm.at[idx_b], rows_v.at[pl.ds(128, 72)], sem)
        c1.wait()
        c2.wait()

        @plsc.parallel_loop(0, S, unroll=8)
        def tok_body(s):
            tt = tt_v[pl.ds(s, L)][0]
            xs = []
            acc_s = None
            acc_q = None
            for j in range(NH):
                sl = pl.ds(j * L, L)
                x = rows_v[s, sl] + pos_v[s, sl] + typ_v[tt, sl]
                xs.append(x)
                acc_s = x if acc_s is None else acc_s + x
                acc_q = x * x if acc_q is None else acc_q + x * x
            mean = _lane_sum(acc_s) * (1.0 / H)
            ex2 = _lane_sum(acc_q) * (1.0 / H)
            inv = _rsqrt(ex2 - mean * mean + 1e-12)
            for j in range(NH):
                sl = pl.ds(j * L, L)
                rows_v[s, sl] = (xs[j] - mean) * inv * g_v[sl] + b_v[sl]
        pltpu.sync_copy(rows_v, out_hbm.at[b])
        return carry

    lax.fori_loop(0, ROWS_PER_W, row_body, 0)


def kernel(input_ids, token_type_ids, word_embeddings, position_embeddings,
           token_type_embeddings, ln_gamma, ln_beta):
    mesh = plsc.VectorSubcoreMesh(core_axis_name="c", subcore_axis_name="s")
    f = pl.kernel(
        _sc_body,
        out_type=jax.ShapeDtypeStruct((B, S, H), jnp.float32),
        mesh=mesh,
        scratch_types=[
            pltpu.VMEM((S, H), jnp.float32),
            pltpu.VMEM((S, H), jnp.float32),
            pltpu.VMEM((T, H), jnp.float32),
            pltpu.VMEM((H,), jnp.float32),
            pltpu.VMEM((H,), jnp.float32),
            pltpu.VMEM((128,), jnp.int32),
            pltpu.VMEM((72,), jnp.int32),
            pltpu.VMEM((S + L,), jnp.int32),
            pltpu.SemaphoreType.DMA,
        ],
    )
    return f(input_ids.astype(jnp.int32).reshape(-1),
             token_type_ids.astype(jnp.int32).reshape(-1),
             word_embeddings, position_embeddings, token_type_embeddings,
             ln_gamma, ln_beta)


# speedup vs baseline: 10.4611x; 1.4823x over previous
import jax
import jax.numpy as jnp
from jax import lax
from jax.experimental import pallas as pl
from jax.experimental.pallas import tpu as pltpu
from jax.experimental.pallas import tpu_sc as plsc

B, S, V, H, P, T = 4096, 200, 100000, 128, 512, 2

NC, NS, L = 2, 16, 16
NW = NC * NS
ROWS_PER_W = B // NW
NH = H // L
TOK_W = ROWS_PER_W * S

C0, C1 = 104, 96
NCHUNK = 2 * ROWS_PER_W
NBUF = 4
INNER = 8
UNROLL = 8


def _lane_sum(x):
    lanes = jnp.arange(L, dtype=jnp.int32)
    dnums = lax.GatherDimensionNumbers(
        offset_dims=(), collapsed_slice_dims=(0,), start_index_map=(0,))
    for d in (8, 4, 2, 1):
        x = x + lax.gather(x, (lanes ^ d)[:, None], dnums, slice_sizes=(1,),
                           mode=lax.GatherScatterMode.PROMISE_IN_BOUNDS)
    return x


def _rsqrt(x):
    i = lax.bitcast_convert_type(x, jnp.int32)
    i = jnp.int32(0x5F3759DF) - (i >> 1)
    y = lax.bitcast_convert_type(i, jnp.float32)
    for _ in range(3):
        y = y * (1.5 - 0.5 * x * y * y)
    return y


def _sc_body(ids_hbm, tt_hbm, word_hbm, pos_hbm, typ_hbm, g_hbm, b_hbm,
             out_hbm, buf0, buf1, buf2, buf3, ids_v, tt_cb, pos_v, typ_v,
             g_v, b_v, g_sems, o_sems, t_sems):
    wid = lax.axis_index("s") * NC + lax.axis_index("c")
    b0 = wid * ROWS_PER_W
    tok0 = b0 * S
    bufs = (buf0, buf1, buf2, buf3)

    pltpu.sync_copy(pos_hbm.at[pl.ds(0, S)], pos_v)
    pltpu.sync_copy(typ_hbm, typ_v)
    pltpu.sync_copy(g_hbm, g_v)
    pltpu.sync_copy(b_hbm, b_v)
    pltpu.sync_copy(ids_hbm.at[pl.ds(tok0, TOK_W)], ids_v)

    def chunk_geom(m):
        off = 0 if m % 2 == 0 else C0
        ln = C0 if m % 2 == 0 else C1
        return off, ln

    def start_gather(c_outer, m):
        off, ln = chunk_geom(m)
        k = m % NBUF
        row = c_outer * (INNER // 2) + m // 2
        p = row * S + off
        pltpu.async_copy(
            word_hbm.at[ids_v.at[pl.ds(p, ln)]],
            bufs[k].at[pl.ds(0, ln)], g_sems.at[k])
        pltpu.async_copy(
            tt_hbm.at[pl.ds(tok0 + p, ln)],
            tt_cb.at[k, pl.ds(0, ln)], t_sems.at[k])

    start_gather(0, 0)
    start_gather(0, 1)

    def outer_body(t, carry):
        for m in range(INNER):
            off, ln = chunk_geom(m)
            k = m % NBUF
            k2 = (m + 2) % NBUF
            _, ln2 = chunk_geom(m + 2)
            c = t * INNER + m
            row = t * (INNER // 2) + m // 2
            b = b0 + row
            p0 = row * S + off

            pltpu.make_async_copy(
                word_hbm.at[ids_v.at[pl.ds(p0, ln)]],
                bufs[k].at[pl.ds(0, ln)], g_sems.at[k]).wait()
            pltpu.make_async_copy(
                tt_hbm.at[pl.ds(tok0 + p0, ln)],
                tt_cb.at[k, pl.ds(0, ln)], t_sems.at[k]).wait()

            @pl.when(c >= 2)
            def _():
                pltpu.make_async_copy(
                    bufs[k2].at[pl.ds(0, ln2)],
                    out_hbm.at[b, pl.ds(0, ln2)], o_sems.at[k2]).wait()

            @pl.when(c + 2 < NCHUNK)
            def _():
                off2 = 0 if (m + 2) % 2 == 0 else C0
                row2 = (c + 2) // 2
                p2 = row2 * S + off2
                pltpu.async_copy(
                    word_hbm.at[ids_v.at[pl.ds(p2, ln2)]],
                    bufs[k2].at[pl.ds(0, ln2)], g_sems.at[k2])
                pltpu.async_copy(
                    tt_hbm.at[pl.ds(tok0 + p2, ln2)],
                    tt_cb.at[k2, pl.ds(0, ln2)], t_sems.at[k2])

            buf = bufs[k]

            @plsc.parallel_loop(0, ln, unroll=UNROLL)
            def tok_body(s):
                tt = tt_cb[k, pl.ds(s, L)][0]
                xs = []
                acc_s = None
                acc_q = None
                for j in range(NH):
                    sl = pl.ds(j * L, L)
                    x = buf[s, sl] + pos_v[off + s, sl] + typ_v[tt, sl]
                    xs.append(x)
                    acc_s = x if acc_s is None else acc_s + x
                    acc_q = x * x if acc_q is None else acc_q + x * x
                mean = _lane_sum(acc_s) * (1.0 / H)
                ex2 = _lane_sum(acc_q) * (1.0 / H)
                inv = _rsqrt(ex2 - mean * mean + 1e-12)
                for j in range(NH):
                    sl = pl.ds(j * L, L)
                    buf[s, sl] = (xs[j] - mean) * inv * g_v[sl] + b_v[sl]

            pltpu.async_copy(
                buf.at[pl.ds(0, ln)],
                out_hbm.at[b, pl.ds(off, ln)], o_sems.at[k])
        return carry

    lax.fori_loop(0, NCHUNK // INNER, outer_body, 0)

    for m in (INNER - 2, INNER - 1):
        off, ln = chunk_geom(m)
        k = m % NBUF
        pltpu.make_async_copy(
            bufs[k].at[pl.ds(0, ln)],
            out_hbm.at[b0, pl.ds(off, ln)], o_sems.at[k]).wait()


def kernel(input_ids, token_type_ids, word_embeddings, position_embeddings,
           token_type_embeddings, ln_gamma, ln_beta):
    mesh = plsc.VectorSubcoreMesh(core_axis_name="c", subcore_axis_name="s")
    f = pl.kernel(
        _sc_body,
        out_type=jax.ShapeDtypeStruct((B, S, H), jnp.float32),
        mesh=mesh,
        scratch_types=[
            pltpu.VMEM((C0, H), jnp.float32),
            pltpu.VMEM((C0, H), jnp.float32),
            pltpu.VMEM((C0, H), jnp.float32),
            pltpu.VMEM((C0, H), jnp.float32),
            pltpu.VMEM((TOK_W,), jnp.int32),
            pltpu.VMEM((NBUF, C0 + L), jnp.int32),
            pltpu.VMEM((S, H), jnp.float32),
            pltpu.VMEM((T, H), jnp.float32),
            pltpu.VMEM((H,), jnp.float32),
            pltpu.VMEM((H,), jnp.float32),
            pltpu.SemaphoreType.DMA((NBUF,)),
            pltpu.SemaphoreType.DMA((NBUF,)),
            pltpu.SemaphoreType.DMA((NBUF,)),
        ],
    )
    return f(input_ids.astype(jnp.int32).reshape(-1),
             token_type_ids.astype(jnp.int32).reshape(-1),
             word_embeddings, position_embeddings, token_type_embeddings,
             ln_gamma, ln_beta)


# speedup vs baseline: 11.5438x; 1.1035x over previous
import jax
import jax.numpy as jnp
from jax import lax
from jax.experimental import pallas as pl
from jax.experimental.pallas import tpu as pltpu
from jax.experimental.pallas import tpu_sc as plsc

B, S, V, H, P, T = 4096, 200, 100000, 128, 512, 2

NC, NS, L = 2, 16, 16
NW = NC * NS
ROWS_PER_W = B // NW
NH = H // L
TOK_W = ROWS_PER_W * S

C0, C1 = 104, 96
NCHUNK = 2 * ROWS_PER_W
NBUF = 4
INNER = 8
UNROLL = 8


def _lane_sum(x):
    lanes = jnp.arange(L, dtype=jnp.int32)
    dnums = lax.GatherDimensionNumbers(
        offset_dims=(), collapsed_slice_dims=(0,), start_index_map=(0,))
    for d in (8, 4, 2, 1):
        x = x + lax.gather(x, (lanes ^ d)[:, None], dnums, slice_sizes=(1,),
                           mode=lax.GatherScatterMode.PROMISE_IN_BOUNDS)
    return x


def _rsqrt(x):
    i = lax.bitcast_convert_type(x, jnp.int32)
    i = jnp.int32(0x5F3759DF) - (i >> 1)
    y = lax.bitcast_convert_type(i, jnp.float32)
    for _ in range(3):
        y = y * (1.5 - 0.5 * x * y * y)
    return y


def _sc_body(ids_hbm, tt_hbm, word_hbm, pos_hbm, typ_hbm, g_hbm, b_hbm,
             out_hbm, buf0, buf1, buf2, buf3, ids_cb, tt_cb, pt_v, typ_v,
             g_v, b_v, g_sems, o_sems, i_sems, t_sems):
    wid = lax.axis_index("s") * NC + lax.axis_index("c")
    b0 = wid * ROWS_PER_W
    tok0 = b0 * S
    bufs = (buf0, buf1, buf2, buf3)

    pltpu.sync_copy(pos_hbm.at[pl.ds(0, S)], pt_v.at[pl.ds(0, S)])
    pltpu.sync_copy(typ_hbm, typ_v)
    pltpu.sync_copy(g_hbm, g_v)
    pltpu.sync_copy(b_hbm, b_v)

    @plsc.parallel_loop(0, S, unroll=4)
    def build_pt(s):
        for j in range(NH):
            sl = pl.ds(j * L, L)
            p = pt_v[s, sl]
            pt_v[S + s, sl] = p + typ_v[1, sl]
            pt_v[s, sl] = p + typ_v[0, sl]

    def chunk_geom(m):
        off = 0 if m % 2 == 0 else C0
        ln = C0 if m % 2 == 0 else C1
        return off, ln

    def tok_start(c_outer, m):
        off, _ = chunk_geom(m)
        row = c_outer * (INNER // 2) + m // 2
        return row * S + off

    def issue_ids(c_outer, m, k):
        _, ln = chunk_geom(m)
        p = tok_start(c_outer, m)
        pltpu.async_copy(ids_hbm.at[pl.ds(tok0 + p, ln)],
                         ids_cb.at[k, pl.ds(0, ln)], i_sems.at[k])
        pltpu.async_copy(tt_hbm.at[pl.ds(tok0 + p, ln)],
                         tt_cb.at[k, pl.ds(0, ln)], t_sems.at[k])

    def wait_ids(m, k):
        _, ln = chunk_geom(m)
        pltpu.make_async_copy(ids_hbm.at[pl.ds(tok0, ln)],
                              ids_cb.at[k, pl.ds(0, ln)], i_sems.at[k]).wait()

    def issue_gather(m, k):
        _, ln = chunk_geom(m)
        pltpu.async_copy(word_hbm.at[ids_cb.at[k, pl.ds(0, ln)]],
                         bufs[k].at[pl.ds(0, ln)], g_sems.at[k])

    for m in range(NBUF):
        issue_ids(0, m, m)
    for m in range(2):
        wait_ids(m, m)
        issue_gather(m, m)

    def outer_body(t, carry):
        for m in range(INNER):
            off, ln = chunk_geom(m)
            k = m % NBUF
            k2 = (m + 2) % NBUF
            _, ln2 = chunk_geom(m + 2)
            c = t * INNER + m
            row = t * (INNER // 2) + m // 2
            b = b0 + row
            p0 = row * S + off

            pltpu.make_async_copy(
                word_hbm.at[ids_cb.at[k, pl.ds(0, ln)]],
                bufs[k].at[pl.ds(0, ln)], g_sems.at[k]).wait()
            pltpu.make_async_copy(
                tt_hbm.at[pl.ds(tok0, ln)],
                tt_cb.at[k, pl.ds(0, ln)], t_sems.at[k]).wait()

            @pl.when(c >= 2)
            def _():
                pltpu.make_async_copy(
                    bufs[k2].at[pl.ds(0, ln2)],
                    out_hbm.at[b, pl.ds(0, ln2)], o_sems.at[k2]).wait()

            @pl.when(c + 2 < NCHUNK)
            def _():
                wait_ids(m + 2, k2)
                issue_gather(m + 2, k2)

            buf = bufs[k]

            @plsc.parallel_loop(0, ln, unroll=UNROLL)
            def tok_body(s):
                tt = tt_cb[k, pl.ds(s, L)][0]
                ptrow = tt * S + (off + s)
                xs = []
                acc_s = None
                acc_q = None
                for j in range(NH):
                    sl = pl.ds(j * L, L)
                    x = buf[s, sl] + pt_v[ptrow, sl]
                    xs.append(x)
                    acc_s = x if acc_s is None else acc_s + x
                    acc_q = x * x if acc_q is None else acc_q + x * x
                mean = _lane_sum(acc_s) * (1.0 / H)
                ex2 = _lane_sum(acc_q) * (1.0 / H)
                inv = _rsqrt(ex2 - mean * mean + 1e-12)
                for j in range(NH):
                    sl = pl.ds(j * L, L)
                    buf[s, sl] = (xs[j] - mean) * inv * g_v[sl] + b_v[sl]

            pltpu.async_copy(
                buf.at[pl.ds(0, ln)],
                out_hbm.at[b, pl.ds(off, ln)], o_sems.at[k])

            @pl.when(c + NBUF < NCHUNK)
            def _():
                row4 = (c + NBUF) // 2
                off4, ln4 = chunk_geom(m)
                p4 = row4 * S + off4
                pltpu.async_copy(ids_hbm.at[pl.ds(tok0 + p4, ln4)],
                                 ids_cb.at[k, pl.ds(0, ln4)], i_sems.at[k])
                pltpu.async_copy(tt_hbm.at[pl.ds(tok0 + p4, ln4)],
                                 tt_cb.at[k, pl.ds(0, ln4)], t_sems.at[k])
        return carry

    lax.fori_loop(0, NCHUNK // INNER, outer_body, 0)

    for m in (INNER - 2, INNER - 1):
        off, ln = chunk_geom(m)
        k = m % NBUF
        pltpu.make_async_copy(
            bufs[k].at[pl.ds(0, ln)],
            out_hbm.at[b0, pl.ds(off, ln)], o_sems.at[k]).wait()


def kernel(input_ids, token_type_ids, word_embeddings, position_embeddings,
           token_type_embeddings, ln_gamma, ln_beta):
    mesh = plsc.VectorSubcoreMesh(core_axis_name="c", subcore_axis_name="s")
    f = pl.kernel(
        _sc_body,
        out_type=jax.ShapeDtypeStruct((B, S, H), jnp.float32),
        mesh=mesh,
        scratch_types=[
            pltpu.VMEM((C0, H), jnp.float32),
            pltpu.VMEM((C0, H), jnp.float32),
            pltpu.VMEM((C0, H), jnp.float32),
            pltpu.VMEM((C0, H), jnp.float32),
            pltpu.VMEM((NBUF, C0 + 8), jnp.int32),
            pltpu.VMEM((NBUF, C0 + L), jnp.int32),
            pltpu.VMEM((2 * S, H), jnp.float32),
            pltpu.VMEM((T, H), jnp.float32),
            pltpu.VMEM((H,), jnp.float32),
            pltpu.VMEM((H,), jnp.float32),
            pltpu.SemaphoreType.DMA((NBUF,)),
            pltpu.SemaphoreType.DMA((NBUF,)),
            pltpu.SemaphoreType.DMA((NBUF,)),
            pltpu.SemaphoreType.DMA((NBUF,)),
        ],
    )
    return f(input_ids.astype(jnp.int32).reshape(-1),
             token_type_ids.astype(jnp.int32).reshape(-1),
             word_embeddings, position_embeddings, token_type_embeddings,
             ln_gamma, ln_beta)
